# baseline (speedup 1.0000x reference)
import functools
import math

import jax
import jax.numpy as jnp
from jax import lax
from jax.experimental import pallas as pl
from jax.experimental.pallas import tpu as pltpu
from jax.experimental.pallas import tpu_sc as plsc

D_MODEL = 64
SCALE = math.sqrt(D_MODEL)

NUM_CORES = 2
NUM_SUBCORES = 16
NW = NUM_CORES * NUM_SUBCORES

IDXW = 128
CHUNK = 1024
KS = CHUNK // IDXW


def _build_gather(B: int, V: int):
    assert B % (NW * CHUNK) == 0
    bpw = B // NW
    nchunk = bpw // CHUNK
    idx_rows_pw = bpw // IDXW

    mesh = plsc.VectorSubcoreMesh(core_axis_name="c", subcore_axis_name="s")

    @functools.partial(
        pl.kernel,
        mesh=mesh,
        out_type=jax.ShapeDtypeStruct((B, D_MODEL), jnp.float32),
        compiler_params=pltpu.CompilerParams(use_tc_tiling_on_sc=False),
        scratch_types=[
            pltpu.VMEM((KS, IDXW), jnp.int32),
            pltpu.VMEM((CHUNK, D_MODEL), jnp.float32),
            pltpu.SemaphoreType.DMA,
        ],
    )
    def gather_kernel(idx_hbm, table_hbm, out_hbm, idx_v, rows_v, gsem):
        cid = lax.axis_index("c")
        sid = lax.axis_index("s")
        wid = sid * NUM_CORES + cid

        def chunk_body(ci, carry):
            irow0 = wid * idx_rows_pw + ci * KS
            pltpu.sync_copy(idx_hbm.at[pl.ds(irow0, KS)], idx_v)
            copies = []
            for j in range(KS):
                copies.append(
                    pltpu.async_copy(
                        table_hbm.at[idx_v.at[j]],
                        rows_v.at[pl.ds(j * IDXW, IDXW)],
                        gsem,
                    )
                )
            for c in copies:
                c.wait()

            def scale_row(r, c2):
                for l in range(D_MODEL // 16):
                    rows_v[r, pl.ds(l * 16, 16)] = (
                        rows_v[r, pl.ds(l * 16, 16)] * SCALE
                    )
                return c2

            lax.fori_loop(0, CHUNK, scale_row, 0)

            out0 = wid * bpw + ci * CHUNK
            pltpu.sync_copy(rows_v, out_hbm.at[pl.ds(out0, CHUNK)])
            return carry

        lax.fori_loop(0, nchunk, chunk_body, 0)

    return gather_kernel


def kernel(x, table):
    B = x.size
    V = table.shape[0]
    idx2d = x.reshape(B // IDXW, IDXW)
    out = _build_gather(B, V)(idx2d, table)
    return out.reshape(x.shape + (D_MODEL,))


# speedup vs baseline: 1.1093x; 1.1093x over previous
import functools
import math

import jax
import jax.numpy as jnp
from jax import lax
from jax.experimental import pallas as pl
from jax.experimental.pallas import tpu as pltpu
from jax.experimental.pallas import tpu_sc as plsc

D_MODEL = 64
SCALE = math.sqrt(D_MODEL)

NUM_CORES = 2
NUM_SUBCORES = 16
NW = NUM_CORES * NUM_SUBCORES

IDXW = 128
CHUNK = 640
KS = CHUNK // IDXW
RUNROLL = 8


def _build_gather(B: int, V: int):
    assert B % (NW * 2 * CHUNK) == 0
    bpw = B // NW
    nchunk = bpw // CHUNK
    idx_rows_pw = bpw // IDXW

    mesh = plsc.VectorSubcoreMesh(core_axis_name="c", subcore_axis_name="s")

    @functools.partial(
        pl.kernel,
        mesh=mesh,
        out_type=jax.ShapeDtypeStruct((B, D_MODEL), jnp.float32),
        compiler_params=pltpu.CompilerParams(use_tc_tiling_on_sc=False),
        scratch_types=[
            pltpu.VMEM((idx_rows_pw, IDXW), jnp.int32),
            pltpu.VMEM((CHUNK, D_MODEL), jnp.float32),
            pltpu.VMEM((CHUNK, D_MODEL), jnp.float32),
            pltpu.SemaphoreType.DMA,
            pltpu.SemaphoreType.DMA,
            pltpu.SemaphoreType.DMA,
            pltpu.SemaphoreType.DMA,
        ],
    )
    def gather_kernel(idx_hbm, table_hbm, out_hbm, idx_v, rows0, rows1,
                      gsem0, gsem1, osem0, osem1):
        cid = lax.axis_index("c")
        sid = lax.axis_index("s")
        wid = sid * NUM_CORES + cid
        rows = (rows0, rows1)
        gsem = (gsem0, gsem1)
        osem = (osem0, osem1)

        pltpu.sync_copy(idx_hbm.at[pl.ds(wid * idx_rows_pw, idx_rows_pw)],
                        idx_v)

        def fire(ci, p):
            for j in range(KS):
                pltpu.async_copy(
                    table_hbm.at[idx_v.at[ci * KS + j]],
                    rows[p].at[pl.ds(j * IDXW, IDXW)],
                    gsem[p],
                )

        def drain_gathers(p):
            for j in range(KS):
                pltpu.make_async_copy(
                    table_hbm.at[idx_v.at[0]],
                    rows[p].at[pl.ds(j * IDXW, IDXW)],
                    gsem[p],
                ).wait()

        def drain_out(p):
            pltpu.make_async_copy(
                rows[p],
                out_hbm.at[pl.ds(0, CHUNK)],
                osem[p],
            ).wait()

        def scale(p):
            buf = rows[p]

            def srows(r, carry):
                base = r * RUNROLL
                for k in range(RUNROLL):
                    for l in range(D_MODEL // 16):
                        buf[base + k, pl.ds(l * 16, 16)] = (
                            buf[base + k, pl.ds(l * 16, 16)] * SCALE
                        )
                return carry

            lax.fori_loop(0, CHUNK // RUNROLL, srows, 0, unroll=False)

        fire(0, 0)

        def pair_body(i, carry):
            for p in (0, 1):
                ci = 2 * i + p
                nci = ci + 1

                @pl.when(nci < nchunk)
                def _():
                    @pl.when(ci >= 1)
                    def _():
                        drain_out(1 - p)
                    fire(nci, 1 - p)

                drain_gathers(p)
                scale(p)
                pltpu.async_copy(
                    rows[p],
                    out_hbm.at[pl.ds(wid * bpw + ci * CHUNK, CHUNK)],
                    osem[p],
                )
            return carry

        lax.fori_loop(0, nchunk // 2, pair_body, 0, unroll=False)
        drain_out(0)
        drain_out(1)

    return gather_kernel


def kernel(x, table):
    B = x.size
    V = table.shape[0]
    idx2d = x.reshape(B // IDXW, IDXW)
    out = _build_gather(B, V)(idx2d, table)
    return out.reshape(x.shape + (D_MODEL,))
